# Initial kernel scaffold; baseline (speedup 1.0000x reference)
import jax
import jax.numpy as jnp
from jax.experimental import pallas as pl


def kernel(x, edge_index, edge_weight, W1, b1, g1, be1, W2, b2, g2, be2, Wb, bb, gb, beb, Wg1, bg1, Wg2, bg2, Wc, bc):
    raise NotImplementedError("write your pallas kernel here")


# speedup vs baseline: 5.2906x; 5.2906x over previous
import functools

import jax
import jax.numpy as jnp
from jax import lax
from jax.experimental import pallas as pl
from jax.experimental.pallas import tpu as pltpu
from jax.experimental.pallas import tpu_sc as plsc

N = 10000
F = 128
NINC = 320000
NC = 2
NS = 16
NW = NC * NS
CHUNK = 128
NCH_W = 80
NPAD = NW * NCH_W * CHUNK
ACC_ROWS = 10240
TROWS = ACC_ROWS // NS
PAD = N
DEGW = 16

_mesh = plsc.VectorSubcoreMesh(core_axis_name="c", subcore_axis_name="s",
                               num_cores=NC, num_subcores=NS)


@functools.partial(
    pl.kernel,
    out_type=jax.ShapeDtypeStruct((NC, ACC_ROWS, F), jnp.float32),
    mesh=_mesh,
    scratch_types=[
        pltpu.VMEM((NCH_W, CHUNK), jnp.int32),
        pltpu.VMEM((NCH_W, CHUNK), jnp.int32),
        pltpu.VMEM((CHUNK, F), jnp.float32),
        pltpu.VMEM_SHARED((ACC_ROWS, F), jnp.float32),
        pltpu.SemaphoreType.DMA,
    ],
)
def _sc_seg_sum(tbl, gidx, sidx, zblk, out, gi, si, rows, acc, sem):
    c = lax.axis_index("c")
    s = lax.axis_index("s")
    w = c * NS + s
    pltpu.sync_copy(gidx.at[pl.ds(w * NCH_W, NCH_W)], gi)
    pltpu.sync_copy(sidx.at[pl.ds(w * NCH_W, NCH_W)], si)
    pltpu.sync_copy(zblk, rows)
    for j in range(TROWS // CHUNK):
        pltpu.sync_copy(rows, acc.at[pl.ds(s * TROWS + j * CHUNK, CHUNK)])
    plsc.subcore_barrier()

    def body(j, carry):
        pltpu.async_copy(tbl.at[gi.at[j]], rows, sem).wait()
        pltpu.sync_copy(rows, acc.at[si.at[j]], add=True)
        return carry

    lax.fori_loop(0, NCH_W, body, 0)
    plsc.subcore_barrier()
    for j in range(TROWS // CHUNK):
        r0 = s * TROWS + j * CHUNK
        pltpu.sync_copy(acc.at[pl.ds(r0, CHUNK)], rows)
        pltpu.sync_copy(rows, out.at[c, pl.ds(r0, CHUNK)])


R = 1000


def _ln_leaky(h, g, b):
    mu = jnp.mean(h, axis=-1, keepdims=True)
    var = jnp.mean((h - mu) ** 2, axis=-1, keepdims=True)
    h = (h - mu) / jnp.sqrt(var + 1e-5) * g + b
    return jnp.where(h >= 0, h, 0.2 * h)


def _tc_pre_body(x_ref, w1_ref, wb_ref, bb_ref, gb_ref, beb_ref,
                 xw1_ref, zb_ref):
    xb = x_ref[...]
    xw1_ref[...] = jnp.dot(xb, w1_ref[...], preferred_element_type=jnp.float32)
    h = jnp.dot(xb, wb_ref[...], preferred_element_type=jnp.float32) + bb_ref[...]
    zb_ref[...] = _ln_leaky(h, gb_ref[...], beb_ref[...])


def _tc_scale_body(sp_ref, bp_ref, m_ref):
    bp = bp_ref[...]
    bcnt = bp[0, :, 1:2] + bp[1, :, 1:2]
    binv = jnp.where(bcnt > 0, 1.0 / jnp.where(bcnt > 0, bcnt, 1.0), 0.0)
    m_ref[...] = binv * (sp_ref[0] + sp_ref[1])


def _tc_z1_body(sp_ref, dp_ref, b1_ref, g1_ref, be1_ref, w2_ref,
                z1_ref, xw2_ref):
    dp = dp_ref[...]
    d = dp[0, :, 0:1] + dp[1, :, 0:1]
    dinv = jnp.where(d > 0, 1.0 / jnp.where(d > 0, d, 1.0), 0.0)
    h = dinv * (sp_ref[0] + sp_ref[1]) + b1_ref[...]
    z1 = _ln_leaky(h, g1_ref[...], be1_ref[...])
    z1_ref[...] = z1
    xw2_ref[...] = jnp.dot(z1, w2_ref[...], preferred_element_type=jnp.float32)


def _tc_tail_body(sp_ref, dp_ref, b2_ref, g2_ref, be2_ref, z1_ref, zb_ref,
                  wg1_ref, bg1_ref, wg2_ref, bg2_ref, wc_ref, bc_ref, out_ref):
    dp = dp_ref[...]
    d = dp[0, :, 0:1] + dp[1, :, 0:1]
    dinv = jnp.where(d > 0, 1.0 / jnp.where(d > 0, d, 1.0), 0.0)
    h = dinv * (sp_ref[0] + sp_ref[1]) + b2_ref[...]
    z2 = _ln_leaky(h, g2_ref[...], be2_ref[...])
    zsp = z1_ref[...] + z2
    zb = zb_ref[...]
    comb = jnp.concatenate([zsp, zb], axis=1)
    se = jax.nn.relu(jnp.dot(comb, wg1_ref[...], preferred_element_type=jnp.float32)
                     + bg1_ref[...])
    gate = jax.nn.sigmoid(jnp.dot(se, wg2_ref[...], preferred_element_type=jnp.float32)
                          + bg2_ref[...])
    fused = gate * zsp + (1.0 - gate) * zb
    out_ref[...] = jnp.dot(fused, wc_ref[...], preferred_element_type=jnp.float32) + bc_ref[...]


def _row_spec(width):
    return pl.BlockSpec((R, width), lambda i: (i, 0))


def _part_spec(width):
    return pl.BlockSpec((NC, R, width), lambda i: (0, i, 0))


def _full_spec(shape):
    nd = len(shape)
    return pl.BlockSpec(shape, lambda i: (0,) * nd)


def kernel(x, edge_index, edge_weight, W1, b1, g1, be1, W2, b2, g2, be2,
           Wb, bb, gb, beb, Wg1, bg1, Wg2, bg2, Wc, bc):
    node = edge_index[0]
    he = edge_index[1]
    padlen = NPAD - NINC
    node_g = jnp.pad(node, (0, padlen)).reshape(NW * NCH_W, CHUNK)
    he_g = jnp.pad(he, (0, padlen)).reshape(NW * NCH_W, CHUNK)
    node_s = jnp.pad(node, (0, padlen), constant_values=PAD).reshape(NW * NCH_W, CHUNK)
    he_s = jnp.pad(he, (0, padlen), constant_values=PAD).reshape(NW * NCH_W, CHUNK)
    zblk = jnp.zeros((CHUNK, F), jnp.float32)
    zblk16 = jnp.zeros((CHUNK, DEGW), jnp.float32)

    tbl16 = jnp.zeros((N, DEGW), jnp.float32)
    tbl16 = tbl16.at[:, 0].set(edge_weight).at[:, 1].set(1.0)
    dpart, bpart = _sc_degrees(tbl16, he_g, node_s, he_s, zblk16)

    grid = (N // R,)
    xw1, zb = pl.pallas_call(
        _tc_pre_body,
        grid=grid,
        in_specs=[_row_spec(F), _full_spec((F, F)), _full_spec((F, F)),
                  _full_spec((F,)), _full_spec((F,)), _full_spec((F,))],
        out_specs=[_row_spec(F), _row_spec(F)],
        out_shape=[jax.ShapeDtypeStruct((N, F), jnp.float32),
                   jax.ShapeDtypeStruct((N, F), jnp.float32)],
    )(x, W1, Wb, bb, gb, beb)

    def sc_pass(tbl, gidx, sidx):
        return _sc_seg_sum(tbl, gidx, sidx, zblk)

    def tc_scale(spart):
        return pl.pallas_call(
            _tc_scale_body,
            grid=grid,
            in_specs=[_part_spec(F), _part_spec(DEGW)],
            out_specs=_row_spec(F),
            out_shape=jax.ShapeDtypeStruct((N, F), jnp.float32),
        )(spart, bpart)

    s1 = sc_pass(xw1, node_g, he_s)
    m1 = tc_scale(s1)
    s2 = sc_pass(m1, he_g, node_s)
    z1, xw2 = pl.pallas_call(
        _tc_z1_body,
        grid=grid,
        in_specs=[_part_spec(F), _part_spec(DEGW), _full_spec((F,)),
                  _full_spec((F,)), _full_spec((F,)), _full_spec((F, F))],
        out_specs=[_row_spec(F), _row_spec(F)],
        out_shape=[jax.ShapeDtypeStruct((N, F), jnp.float32),
                   jax.ShapeDtypeStruct((N, F), jnp.float32)],
    )(s2, dpart, b1, g1, be1, W2)

    s3 = sc_pass(xw2, node_g, he_s)
    m2 = tc_scale(s3)
    s4 = sc_pass(m2, he_g, node_s)

    out = pl.pallas_call(
        _tc_tail_body,
        grid=grid,
        in_specs=[_part_spec(F), _part_spec(DEGW), _full_spec((F,)),
                  _full_spec((F,)), _full_spec((F,)), _row_spec(F), _row_spec(F),
                  _full_spec((2 * F, F // 2)), _full_spec((F // 2,)),
                  _full_spec((F // 2, F)), _full_spec((F,)),
                  _full_spec((F, 1)), _full_spec((1,))],
        out_specs=pl.BlockSpec((R, 1), lambda i: (i, 0)),
        out_shape=jax.ShapeDtypeStruct((N, 1), jnp.float32),
    )(s4, dpart, b2, g2, be2, z1, zb, Wg1, bg1, Wg2, bg2, Wc, bc)
    return out


@functools.partial(
    pl.kernel,
    out_type=(jax.ShapeDtypeStruct((NC, ACC_ROWS, DEGW), jnp.float32),
              jax.ShapeDtypeStruct((NC, ACC_ROWS, DEGW), jnp.float32)),
    mesh=_mesh,
    scratch_types=[
        pltpu.VMEM((NCH_W, CHUNK), jnp.int32),
        pltpu.VMEM((NCH_W, CHUNK), jnp.int32),
        pltpu.VMEM((NCH_W, CHUNK), jnp.int32),
        pltpu.VMEM((CHUNK, DEGW), jnp.float32),
        pltpu.VMEM_SHARED((ACC_ROWS, DEGW), jnp.float32),
        pltpu.VMEM_SHARED((ACC_ROWS, DEGW), jnp.float32),
        pltpu.SemaphoreType.DMA,
    ],
    compiler_params=pltpu.CompilerParams(use_tc_tiling_on_sc=False),
)
def _sc_degrees(tbl, hgidx, nsidx, hsidx, zblk, outd, outb,
                hg, ns_, hs, rows, accd, accb, sem):
    c = lax.axis_index("c")
    s = lax.axis_index("s")
    w = c * NS + s
    pltpu.sync_copy(hgidx.at[pl.ds(w * NCH_W, NCH_W)], hg)
    pltpu.sync_copy(nsidx.at[pl.ds(w * NCH_W, NCH_W)], ns_)
    pltpu.sync_copy(hsidx.at[pl.ds(w * NCH_W, NCH_W)], hs)
    pltpu.sync_copy(zblk, rows)
    for j in range(TROWS // CHUNK):
        z0 = s * TROWS + j * CHUNK
        pltpu.sync_copy(rows, accd.at[pl.ds(z0, CHUNK)])
        pltpu.sync_copy(rows, accb.at[pl.ds(z0, CHUNK)])
    plsc.subcore_barrier()

    def body(j, carry):
        pltpu.async_copy(tbl.at[hg.at[j]], rows, sem).wait()
        pltpu.sync_copy(rows, accd.at[ns_.at[j]], add=True)
        pltpu.sync_copy(rows, accb.at[hs.at[j]], add=True)
        return carry

    lax.fori_loop(0, NCH_W, body, 0)
    plsc.subcore_barrier()
    for j in range(TROWS // CHUNK):
        r0 = s * TROWS + j * CHUNK
        pltpu.sync_copy(accd.at[pl.ds(r0, CHUNK)], rows)
        pltpu.sync_copy(rows, outd.at[c, pl.ds(r0, CHUNK)])
        pltpu.sync_copy(accb.at[pl.ds(r0, CHUNK)], rows)
        pltpu.sync_copy(rows, outb.at[c, pl.ds(r0, CHUNK)])


# speedup vs baseline: 6.0640x; 1.1462x over previous
import functools

import jax
import jax.numpy as jnp
from jax import lax
from jax.experimental import pallas as pl
from jax.experimental.pallas import tpu as pltpu
from jax.experimental.pallas import tpu_sc as plsc

N = 10000
F = 128
FW = 144
NINC = 320000
NC = 2
NS = 16
NW = NC * NS
CH = 64
NCH = 160
NPAD = NW * NCH * CH
ACC_ROWS = 10112
TROWS = ACC_ROWS // NS
PAD = N

_mesh = plsc.VectorSubcoreMesh(core_axis_name="c", subcore_axis_name="s",
                               num_cores=NC, num_subcores=NS)


@functools.partial(
    pl.kernel,
    out_type=jax.ShapeDtypeStruct((NC, ACC_ROWS, FW), jnp.float32),
    mesh=_mesh,
    scratch_types=[
        pltpu.VMEM((NCH, CH), jnp.int32),
        pltpu.VMEM((NCH, CH), jnp.int32),
        pltpu.VMEM((CH, FW), jnp.float32),
        pltpu.VMEM((CH, FW), jnp.float32),
        pltpu.VMEM_SHARED((ACC_ROWS, FW), jnp.float32),
        pltpu.SemaphoreType.DMA,
        pltpu.SemaphoreType.DMA,
    ],
    compiler_params=pltpu.CompilerParams(use_tc_tiling_on_sc=False),
)
def _sc_seg_sum(tbl, gidx, sidx, zblk, out, gi, si, rows_a, rows_b, acc,
                sem_a, sem_b):
    c = lax.axis_index("c")
    s = lax.axis_index("s")
    w = c * NS + s
    pltpu.sync_copy(gidx.at[pl.ds(w * NCH, NCH)], gi)
    pltpu.sync_copy(sidx.at[pl.ds(w * NCH, NCH)], si)
    pltpu.sync_copy(zblk, rows_a)
    for j in range(TROWS // CH):
        pltpu.sync_copy(rows_a, acc.at[pl.ds(s * TROWS + j * CH, CH)])
    pltpu.sync_copy(rows_a.at[pl.ds(0, TROWS % CH)],
                    acc.at[pl.ds(s * TROWS + (TROWS // CH) * CH, TROWS % CH)])
    plsc.subcore_barrier()

    pltpu.async_copy(tbl.at[gi.at[0]], rows_a, sem_a)

    def body(j, carry):
        c0 = 2 * j
        pltpu.async_copy(tbl.at[gi.at[c0 + 1]], rows_b, sem_b)
        pltpu.make_async_copy(tbl.at[gi.at[c0]], rows_a, sem_a).wait()
        pltpu.sync_copy(rows_a, acc.at[si.at[c0]], add=True)
        pltpu.async_copy(tbl.at[gi.at[c0 + 2]], rows_a, sem_a)
        pltpu.make_async_copy(tbl.at[gi.at[c0 + 1]], rows_b, sem_b).wait()
        pltpu.sync_copy(rows_b, acc.at[si.at[c0 + 1]], add=True)
        return carry

    lax.fori_loop(0, NCH // 2 - 1, body, 0)
    pltpu.async_copy(tbl.at[gi.at[NCH - 1]], rows_b, sem_b)
    pltpu.make_async_copy(tbl.at[gi.at[NCH - 2]], rows_a, sem_a).wait()
    pltpu.sync_copy(rows_a, acc.at[si.at[NCH - 2]], add=True)
    pltpu.make_async_copy(tbl.at[gi.at[NCH - 1]], rows_b, sem_b).wait()
    pltpu.sync_copy(rows_b, acc.at[si.at[NCH - 1]], add=True)
    plsc.subcore_barrier()
    for j in range(TROWS // CH):
        r0 = s * TROWS + j * CH
        pltpu.sync_copy(acc.at[pl.ds(r0, CH)], rows_a)
        pltpu.sync_copy(rows_a, out.at[c, pl.ds(r0, CH)])
    r0 = s * TROWS + (TROWS // CH) * CH
    pltpu.sync_copy(acc.at[pl.ds(r0, TROWS % CH)], rows_a.at[pl.ds(0, TROWS % CH)])
    pltpu.sync_copy(rows_a.at[pl.ds(0, TROWS % CH)], out.at[c, pl.ds(r0, TROWS % CH)])


R = 1000


def _ln_leaky(h, g, b):
    mu = jnp.mean(h, axis=-1, keepdims=True)
    var = jnp.mean((h - mu) ** 2, axis=-1, keepdims=True)
    h = (h - mu) / jnp.sqrt(var + 1e-5) * g + b
    return jnp.where(h >= 0, h, 0.2 * h)


def _pack(feats, extra):
    return jnp.concatenate(
        [feats, extra, jnp.zeros((feats.shape[0], FW - F - 1), jnp.float32)],
        axis=1)


def _tc_pre_body(x_ref, w1_ref, wb_ref, bb_ref, gb_ref, beb_ref,
                 t1_ref, zb_ref):
    xb = x_ref[...]
    xw1 = jnp.dot(xb, w1_ref[...], preferred_element_type=jnp.float32)
    t1_ref[...] = _pack(xw1, jnp.ones((xw1.shape[0], 1), jnp.float32))
    h = jnp.dot(xb, wb_ref[...], preferred_element_type=jnp.float32) + bb_ref[...]
    zb_ref[...] = _ln_leaky(h, gb_ref[...], beb_ref[...])


def _tc_scale_body(sp_ref, hw_ref, t_ref):
    q = sp_ref[0] + sp_ref[1]
    bcnt = q[:, F:F + 1]
    binv = jnp.where(bcnt > 0, 1.0 / jnp.where(bcnt > 0, bcnt, 1.0), 0.0)
    t_ref[...] = _pack(binv * q[:, :F], hw_ref[...])


def _tc_z1_body(sp_ref, b1_ref, g1_ref, be1_ref, w2_ref, z1_ref, t3_ref):
    q = sp_ref[0] + sp_ref[1]
    d = q[:, F:F + 1]
    dinv = jnp.where(d > 0, 1.0 / jnp.where(d > 0, d, 1.0), 0.0)
    h = dinv * q[:, :F] + b1_ref[...]
    z1 = _ln_leaky(h, g1_ref[...], be1_ref[...])
    z1_ref[...] = z1
    xw2 = jnp.dot(z1, w2_ref[...], preferred_element_type=jnp.float32)
    t3_ref[...] = _pack(xw2, jnp.ones((xw2.shape[0], 1), jnp.float32))


def _tc_tail_body(sp_ref, b2_ref, g2_ref, be2_ref, z1_ref, zb_ref,
                  wg1_ref, bg1_ref, wg2_ref, bg2_ref, wc_ref, bc_ref, out_ref):
    q = sp_ref[0] + sp_ref[1]
    d = q[:, F:F + 1]
    dinv = jnp.where(d > 0, 1.0 / jnp.where(d > 0, d, 1.0), 0.0)
    h = dinv * q[:, :F] + b2_ref[...]
    z2 = _ln_leaky(h, g2_ref[...], be2_ref[...])
    zsp = z1_ref[...] + z2
    zb = zb_ref[...]
    comb = jnp.concatenate([zsp, zb], axis=1)
    se = jax.nn.relu(jnp.dot(comb, wg1_ref[...], preferred_element_type=jnp.float32)
                     + bg1_ref[...])
    gate = jax.nn.sigmoid(jnp.dot(se, wg2_ref[...], preferred_element_type=jnp.float32)
                          + bg2_ref[...])
    fused = gate * zsp + (1.0 - gate) * zb
    out_ref[...] = jnp.dot(fused, wc_ref[...], preferred_element_type=jnp.float32) + bc_ref[...]


def _row_spec(width):
    return pl.BlockSpec((R, width), lambda i: (i, 0))


def _part_spec(width):
    return pl.BlockSpec((NC, R, width), lambda i: (0, i, 0))


def _full_spec(shape):
    nd = len(shape)
    return pl.BlockSpec(shape, lambda i: (0,) * nd)


def kernel(x, edge_index, edge_weight, W1, b1, g1, be1, W2, b2, g2, be2,
           Wb, bb, gb, beb, Wg1, bg1, Wg2, bg2, Wc, bc):
    node = edge_index[0]
    he = edge_index[1]
    padlen = NPAD - NINC
    node_g = jnp.pad(node, (0, padlen)).reshape(NW * NCH, CH)
    he_g = jnp.pad(he, (0, padlen)).reshape(NW * NCH, CH)
    node_s = jnp.pad(node, (0, padlen), constant_values=PAD).reshape(NW * NCH, CH)
    he_s = jnp.pad(he, (0, padlen), constant_values=PAD).reshape(NW * NCH, CH)
    zblk = jnp.zeros((CH, FW), jnp.float32)
    hw_col = edge_weight.reshape(N, 1)

    grid = (N // R,)
    t1, zb = pl.pallas_call(
        _tc_pre_body,
        grid=grid,
        in_specs=[_row_spec(F), _full_spec((F, F)), _full_spec((F, F)),
                  _full_spec((F,)), _full_spec((F,)), _full_spec((F,))],
        out_specs=[_row_spec(FW), _row_spec(F)],
        out_shape=[jax.ShapeDtypeStruct((N, FW), jnp.float32),
                   jax.ShapeDtypeStruct((N, F), jnp.float32)],
    )(x, W1, Wb, bb, gb, beb)

    def sc_pass(tbl, gidx, sidx):
        return _sc_seg_sum(tbl, gidx, sidx, zblk)

    def tc_scale(spart):
        return pl.pallas_call(
            _tc_scale_body,
            grid=grid,
            in_specs=[_part_spec(FW), _row_spec(1)],
            out_specs=_row_spec(FW),
            out_shape=jax.ShapeDtypeStruct((N, FW), jnp.float32),
        )(spart, hw_col)

    s1 = sc_pass(t1, node_g, he_s)
    t2 = tc_scale(s1)
    s2 = sc_pass(t2, he_g, node_s)
    z1, t3 = pl.pallas_call(
        _tc_z1_body,
        grid=grid,
        in_specs=[_part_spec(FW), _full_spec((F,)), _full_spec((F,)),
                  _full_spec((F,)), _full_spec((F, F))],
        out_specs=[_row_spec(F), _row_spec(FW)],
        out_shape=[jax.ShapeDtypeStruct((N, F), jnp.float32),
                   jax.ShapeDtypeStruct((N, FW), jnp.float32)],
    )(s2, b1, g1, be1, W2)

    s3 = sc_pass(t3, node_g, he_s)
    t4 = tc_scale(s3)
    s4 = sc_pass(t4, he_g, node_s)

    out = pl.pallas_call(
        _tc_tail_body,
        grid=grid,
        in_specs=[_part_spec(FW), _full_spec((F,)), _full_spec((F,)),
                  _full_spec((F,)), _row_spec(F), _row_spec(F),
                  _full_spec((2 * F, F // 2)), _full_spec((F // 2,)),
                  _full_spec((F // 2, F)), _full_spec((F,)),
                  _full_spec((F, 1)), _full_spec((1,))],
        out_specs=pl.BlockSpec((R, 1), lambda i: (i, 0)),
        out_shape=jax.ShapeDtypeStruct((N, 1), jnp.float32),
    )(s4, b2, g2, be2, z1, zb, Wg1, bg1, Wg2, bg2, Wc, bc)
    return out
